# Initial kernel scaffold; baseline (speedup 1.0000x reference)
import jax
import jax.numpy as jnp
from jax.experimental import pallas as pl


def kernel(contract_tokens, emb_table, W, b):
    raise NotImplementedError("write your pallas kernel here")


# speedup vs baseline: 9.2437x; 9.2437x over previous
import functools

import jax
import jax.numpy as jnp
from jax import lax
from jax.experimental import pallas as pl
from jax.experimental.pallas import tpu as pltpu
from jax.experimental.pallas import tpu_sc as plsc

_VOCAB = 100000
_D = 512
_NCL = 30
_B = 4096
_L = 200
_P = 32
_HALF = _L // 2

_info = plsc.get_sparse_core_info()
_NC, _NS = _info.num_cores, _info.num_subcores
_NW = _NC * _NS
_CPW = _B // _NW


def _proj_body(emb_ref, wp_ref, out_ref):
    out_ref[...] = jnp.dot(emb_ref[...], wp_ref[...],
                           preferred_element_type=jnp.float32)


def _project(emb_table, wp):
    rows = 2000
    return pl.pallas_call(
        _proj_body,
        grid=(_VOCAB // rows,),
        in_specs=[
            pl.BlockSpec((rows, _D), lambda i: (i, 0)),
            pl.BlockSpec((_D, _P), lambda i: (0, 0)),
        ],
        out_specs=pl.BlockSpec((rows, _P), lambda i: (i, 0)),
        out_shape=jax.ShapeDtypeStruct((_VOCAB, _P), jnp.float32),
    )(emb_table, wp)


_mesh = plsc.VectorSubcoreMesh(core_axis_name="c", subcore_axis_name="s")


@functools.partial(
    pl.kernel,
    out_type=jax.ShapeDtypeStruct((_B, _P), jnp.float32),
    mesh=_mesh,
    scratch_types=[
        pltpu.VMEM((_CPW, 2, _HALF), jnp.int32),
        pltpu.VMEM((2, _L, _P), jnp.float32),
        pltpu.VMEM((_CPW, _P), jnp.float32),
        pltpu.SemaphoreType.DMA,
        pltpu.SemaphoreType.DMA,
    ],
    compiler_params=pltpu.CompilerParams(use_tc_tiling_on_sc=False),
)
def _pool_kernel(tok_hbm, proj_hbm, out_hbm, idx_v, rows_v, acc_v, sem0, sem1):
    wid = lax.axis_index("s") * _NC + lax.axis_index("c")
    base = wid * _CPW
    pltpu.sync_copy(tok_hbm.at[pl.ds(base, _CPW)], idx_v)
    sems = (sem0, sem1)

    def fire(b, slot):
        pltpu.async_copy(proj_hbm.at[idx_v.at[b, 0]],
                         rows_v.at[slot, pl.ds(0, _HALF)], sems[slot])
        pltpu.async_copy(proj_hbm.at[idx_v.at[b, 1]],
                         rows_v.at[slot, pl.ds(_HALF, _HALF)], sems[slot])

    def wait_slot(slot):
        pltpu.make_async_copy(proj_hbm.at[pl.ds(0, _L)],
                              rows_v.at[slot], sems[slot]).wait()

    def accumulate(b, slot):
        def acc_body(i, carry):
            accs = list(carry)
            for j in range(8):
                r = i * 8 + j
                k = j % 4
                accs[k] = accs[k] + rows_v[slot, r, 0:16]
                accs[4 + k] = accs[4 + k] + rows_v[slot, r, 16:32]
            return tuple(accs)

        z = jnp.zeros((16,), jnp.float32)
        accs = lax.fori_loop(0, _L // 8, acc_body, (z,) * 8)
        scale = jnp.float32(1.0 / _L)
        acc_v[b, 0:16] = ((accs[0] + accs[1]) + (accs[2] + accs[3])) * scale
        acc_v[b, 16:32] = ((accs[4] + accs[5]) + (accs[6] + accs[7])) * scale

    fire(0, 0)
    fire(1, 1)

    def pair_body(p, _):
        b0 = 2 * p
        wait_slot(0)
        accumulate(b0, 0)

        @pl.when(p < _CPW // 2 - 1)
        def _():
            fire(b0 + 2, 0)

        wait_slot(1)
        accumulate(b0 + 1, 1)

        @pl.when(p < _CPW // 2 - 1)
        def _():
            fire(b0 + 3, 1)

        return 0

    lax.fori_loop(0, _CPW // 2, pair_body, 0)
    pltpu.sync_copy(acc_v, out_hbm.at[pl.ds(base, _CPW)])


def _head_body(pooled_ref, bvec_ref, probs_ref, risk_ref):
    x = pooled_ref[...]
    logits = x + bvec_ref[...]
    m = jnp.max(logits, axis=-1, keepdims=True)
    e = jnp.exp(logits - m)
    s = jnp.sum(e, axis=-1, keepdims=True)
    probs_ref[...] = (e / s)[:, :_NCL]
    risk_ref[...] = 1.0 / (1.0 + jnp.exp(-x[:, 30:31]))


def _head(pooled, bvec):
    return pl.pallas_call(
        _head_body,
        out_shape=(
            jax.ShapeDtypeStruct((_B, _NCL), jnp.float32),
            jax.ShapeDtypeStruct((_B, 1), jnp.float32),
        ),
    )(pooled, bvec)


def kernel(contract_tokens, emb_table, W, b):
    tokens = contract_tokens.astype(jnp.int32).reshape(_B, 2, _HALF)
    e0 = jnp.zeros((_D, 2), jnp.float32).at[0, 0].set(1.0)
    wp = jnp.concatenate([W, e0], axis=1)
    bvec = jnp.concatenate(
        [b, jnp.full((2,), -1e30, jnp.float32)]).reshape(1, _P)
    proj = _project(emb_table, wp)
    pooled = _pool_kernel(tokens, proj)
    clause_types, risk_score = _head(pooled, bvec)
    return (clause_types, risk_score)
